# baseline (speedup 1.0000x reference)
import functools

import jax
import jax.numpy as jnp
from jax import lax
from jax.experimental import pallas as pl
from jax.experimental.pallas import tpu as pltpu
from jax.experimental.pallas import tpu_sc as plsc

B = 16384
B2 = B // 2
H = 64
V = 100000

_info = plsc.get_sparse_core_info()
_NC = _info.num_cores
_NS = _info.num_subcores
_NW = _NC * _NS
_HPW = H // _NW

_B2 = 66816
_B2LN = 33152
_TAIL = 128
_TOFF = _B2LN - 96
_BANDS = ((0, 33408, 33408), (33408, 33408, 33408), (_B2, _B2LN, V - _B2))
_BMAX = 33408


def _gather_one(idx_v, sbuf, base, bound, k):
    iv = idx_v[pl.ds(k, 16)]
    d = iv - base
    m = plsc.bitcast(d, jnp.uint32) < jnp.uint32(bound)
    v = plsc.load_gather(sbuf, [d], mask=m)
    return jnp.where(m, v, 0.0)


def _band_gather(idx_v, sbuf, acc_v, base, bound, first):
    @plsc.parallel_loop(0, B, 16, unroll=4)
    def _(k):
        val = _gather_one(idx_v, sbuf, base, bound, k)
        if first:
            acc_v[pl.ds(k, 16)] = val
        else:
            plsc.addupdate(acc_v.at[pl.ds(k, 16)], val)


def _band_gather_final(idx_v, sbuf, acc_v, obuf, base, bound):
    @plsc.parallel_loop(0, B, 32, unroll=2)
    def _(k):
        va = _gather_one(idx_v, sbuf, base, bound, k) + acc_v[pl.ds(k, 16)]
        vb = (_gather_one(idx_v, sbuf, base, bound, k + 16)
              + acc_v[pl.ds(k + 16, 16)])
        packed = plsc.pack(va, vb, format=plsc.PackFormat.INTERLEAVED)
        obuf[pl.ds(lax.div(k, 2), 16)] = plsc.bitcast(packed, jnp.int32)


def _sc_gather_body(tabT_p, tabT_n, tail_p, tail_n, idx_p_hbm, idx_n_hbm,
                    out_p, out_n, idx_v, sb0, sb1, acc_v, obuf, sidx,
                    s0, s1, so):
    sid = lax.axis_index("s")
    wid = sid * _NC + lax.axis_index("c")
    sbufs = (sb0, sb1)
    sems = (s0, s1)

    units = []
    for tab, tail, ih, oh in ((tabT_p, tail_p, idx_p_hbm, out_p),
                              (tabT_n, tail_n, idx_n_hbm, out_n)):
        for j in range(_HPW):
            for band in range(3):
                units.append((tab, tail, ih, oh, j, band))
    nu = len(units)

    def slice_dma(u, bi):
        tab, tail, ih, oh, j, band = units[u]
        h = wid * _HPW + j
        off, ln, _ = _BANDS[band]
        cs = [pltpu.async_copy(tab.at[h, pl.ds(off, ln)],
                               sbufs[bi].at[pl.ds(0, ln)], sems[bi])]
        if band == 2:
            toff = pl.multiple_of(h * _TAIL, _TAIL)
            cs.append(pltpu.async_copy(tail.at[pl.ds(toff, _TAIL)],
                                       sbufs[bi].at[pl.ds(_TOFF, _TAIL)],
                                       sems[bi]))
        return cs

    cur = slice_dma(0, 0)

    @pl.when(sid == 0)
    def _():
        pltpu.sync_copy(idx_p_hbm, sidx.at[0])
        pltpu.sync_copy(idx_n_hbm, sidx.at[1])

    plsc.subcore_barrier()
    pltpu.sync_copy(sidx.at[0], idx_v)
    outcopy = None
    for u in range(nu):
        tab, tail, ih, oh, j, band = units[u]
        bi = u % 2
        nxt = slice_dma(u + 1, 1 - bi) if u + 1 < nu else None
        for c in cur:
            c.wait()
        if u == nu // 2:
            pltpu.sync_copy(sidx.at[1], idx_v)
        off, _, bound = _BANDS[band]
        if band < 2:
            _band_gather(idx_v, sbufs[bi], acc_v, off, bound, band == 0)
        else:
            if outcopy is not None:
                outcopy.wait()
            _band_gather_final(idx_v, sbufs[bi], acc_v, obuf, off, bound)
            outcopy = pltpu.async_copy(obuf, oh.at[wid * _HPW + j], so)
        cur = nxt
    if outcopy is not None:
        outcopy.wait()


_sc_gather = functools.partial(
    pl.kernel,
    mesh=plsc.VectorSubcoreMesh(core_axis_name="c", subcore_axis_name="s"),
    out_type=[
        jax.ShapeDtypeStruct((H, B2), jnp.int32),
        jax.ShapeDtypeStruct((H, B2), jnp.int32),
    ],
    scratch_types=[
        pltpu.VMEM((B,), jnp.int32),
        pltpu.VMEM((_BMAX,), jnp.float32),
        pltpu.VMEM((_BMAX,), jnp.float32),
        pltpu.VMEM((B,), jnp.float32),
        pltpu.VMEM((B2,), jnp.int32),
        pltpu.VMEM_SHARED((2, B), jnp.int32),
        pltpu.SemaphoreType.DMA,
        pltpu.SemaphoreType.DMA,
        pltpu.SemaphoreType.DMA,
    ],
    compiler_params=pltpu.CompilerParams(needs_layout_passes=False),
)(_sc_gather_body)


_BN = 4096


def _unpack2(w):
    lo = lax.bitcast_convert_type(lax.shift_left(w, 16), jnp.float32)
    hi = lax.bitcast_convert_type(
        lax.bitwise_and(w, jnp.int32(-65536)), jnp.float32)
    return lo, hi


def _mlp_body(pt_ref, nt_ref, w1t_ref, b1_ref, w2_ref, b2_ref,
              w3t_ref, b3_ref, oa_ref, ob_ref):
    mm = functools.partial(jnp.dot, preferred_element_type=jnp.float32)
    w1t = w1t_ref[...]
    b1 = b1_ref[...].T
    b2 = b2_ref[...].T
    pa, pb = _unpack2(pt_ref[...])
    na, nb = _unpack2(nt_ref[...])
    for p, n, o_ref in ((pa, na, oa_ref), (pb, nb, ob_ref)):
        h = mm(w1t[:, :H], p) + mm(w1t[:, H:], n)
        h = jnp.maximum(h + b1, 0.0)
        h2 = lax.dot_general(w2_ref[...], h,
                             dimension_numbers=(((0,), (0,)), ((), ())),
                             preferred_element_type=jnp.float32)
        h2 = jnp.maximum(h2 + b2, 0.0)
        o_ref[...] = mm(w3t_ref[...], h2) + b3_ref[...]


def _mlp(pt, nt, w1t, b1, w2, b2, w3t, b3):
    grid = (B2 // _BN,)
    return pl.pallas_call(
        _mlp_body,
        grid=grid,
        in_specs=[
            pl.BlockSpec((H, _BN), lambda i: (0, i)),
            pl.BlockSpec((H, _BN), lambda i: (0, i)),
            pl.BlockSpec((H, 2 * H), lambda i: (0, 0)),
            pl.BlockSpec((1, H), lambda i: (0, 0)),
            pl.BlockSpec((H, H), lambda i: (0, 0)),
            pl.BlockSpec((1, H), lambda i: (0, 0)),
            pl.BlockSpec((1, H), lambda i: (0, 0)),
            pl.BlockSpec((1, 1), lambda i: (0, 0)),
        ],
        out_specs=[
            pl.BlockSpec((1, _BN), lambda i: (0, i)),
            pl.BlockSpec((1, _BN), lambda i: (0, i)),
        ],
        out_shape=[
            jax.ShapeDtypeStruct((1, B2), jnp.float32),
            jax.ShapeDtypeStruct((1, B2), jnp.float32),
        ],
    )(pt, nt, w1t, b1, w2, b2, w3t, b3)


def kernel(x, emb_proton, emb_neutron, W1, b1, W2, b2, W3, b3):
    x = x.astype(jnp.int32)
    idx_p = x[:, 0]
    idx_n = x[:, 1]
    tp = emb_proton.T
    tn = emb_neutron.T
    pt, nt = _sc_gather(tp, tn, tp[:, V - _TAIL:].reshape(-1),
                        tn[:, V - _TAIL:].reshape(-1), idx_p, idx_n)
    oa, ob = _mlp(pt, nt, W1.T, b1.reshape(1, H), W2, b2.reshape(1, H),
                  W3.reshape(1, H), b3.reshape(1, 1))
    out = jnp.concatenate(
        [oa.reshape(B // 32, 16), ob.reshape(B // 32, 16)], axis=1)
    return out.reshape(B, 1)


# speedup vs baseline: 1.0447x; 1.0447x over previous
import functools

import jax
import jax.numpy as jnp
from jax import lax
from jax.experimental import pallas as pl
from jax.experimental.pallas import tpu as pltpu
from jax.experimental.pallas import tpu_sc as plsc

B = 16384
H = 64
V = 100000

_info = plsc.get_sparse_core_info()
_NC = _info.num_cores
_NS = _info.num_subcores
_NW = _NC * _NS
_HPW = H // _NW

_B2 = 66816
_B2LN = 33152
_TAIL = 128
_TOFF = _B2LN - 96
_BANDS = ((0, 33408, 33408), (33408, 33408, 33408), (_B2, _B2LN, V - _B2))
_BMAX = 33408


def _band_gather(idx_v, sbuf, out_v, base, bound, first):
    @plsc.parallel_loop(0, B, 16, unroll=4)
    def _(k):
        iv = idx_v[pl.ds(k, 16)]
        d = iv - base
        m = plsc.bitcast(d, jnp.uint32) < jnp.uint32(bound)
        v = plsc.load_gather(sbuf, [d], mask=m)
        val = jnp.where(m, v, 0.0)
        if first:
            out_v[pl.ds(k, 16)] = val
        else:
            plsc.addupdate(out_v.at[pl.ds(k, 16)], val)


def _sc_gather_body(tabT_p, tabT_n, tail_p, tail_n, idx_p_hbm, idx_n_hbm,
                    out_p, out_n, idx_v, sb0, sb1, ov0, ov1, sidx, s0, s1,
                    so0, so1):
    sid = lax.axis_index("s")
    wid = sid * _NC + lax.axis_index("c")
    sbufs = (sb0, sb1)
    sems = (s0, s1)
    outs = (ov0, ov1)
    osems = (so0, so1)

    units = []
    for tab, tail, ih, oh in ((tabT_p, tail_p, idx_p_hbm, out_p),
                              (tabT_n, tail_n, idx_n_hbm, out_n)):
        for j in range(_HPW):
            for band in range(3):
                units.append((tab, tail, ih, oh, j, band))
    nu = len(units)

    def slice_dma(u, bi):
        tab, tail, ih, oh, j, band = units[u]
        h = wid * _HPW + j
        off, ln, _ = _BANDS[band]
        cs = [pltpu.async_copy(tab.at[h, pl.ds(off, ln)],
                               sbufs[bi].at[pl.ds(0, ln)], sems[bi])]
        if band == 2:
            toff = pl.multiple_of(h * _TAIL, _TAIL)
            cs.append(pltpu.async_copy(tail.at[pl.ds(toff, _TAIL)],
                                       sbufs[bi].at[pl.ds(_TOFF, _TAIL)],
                                       sems[bi]))
        return cs

    cur = slice_dma(0, 0)

    @pl.when(sid == 0)
    def _():
        pltpu.sync_copy(idx_p_hbm, sidx.at[0])
        pltpu.sync_copy(idx_n_hbm, sidx.at[1])

    plsc.subcore_barrier()
    pltpu.sync_copy(sidx.at[0], idx_v)
    outcopies = [None, None]
    for u in range(nu):
        tab, tail, ih, oh, j, band = units[u]
        bi = u % 2
        nxt = slice_dma(u + 1, 1 - bi) if u + 1 < nu else None
        for c in cur:
            c.wait()
        if u == nu // 2:
            pltpu.sync_copy(sidx.at[1], idx_v)
        oi = j % 2
        if band == 0 and outcopies[oi] is not None:
            outcopies[oi].wait()
            outcopies[oi] = None
        off, _, bound = _BANDS[band]
        _band_gather(idx_v, sbufs[bi], outs[oi], off, bound, band == 0)
        if band == 2:
            outcopies[oi] = pltpu.async_copy(
                outs[oi], oh.at[wid * _HPW + j], osems[oi])
        cur = nxt
    for oc in outcopies:
        if oc is not None:
            oc.wait()


_sc_gather = functools.partial(
    pl.kernel,
    mesh=plsc.VectorSubcoreMesh(core_axis_name="c", subcore_axis_name="s"),
    out_type=[
        jax.ShapeDtypeStruct((H, B), jnp.float32),
        jax.ShapeDtypeStruct((H, B), jnp.float32),
    ],
    scratch_types=[
        pltpu.VMEM((B,), jnp.int32),
        pltpu.VMEM((_BMAX,), jnp.float32),
        pltpu.VMEM((_BMAX,), jnp.float32),
        pltpu.VMEM((B,), jnp.float32),
        pltpu.VMEM((B,), jnp.float32),
        pltpu.VMEM_SHARED((2, B), jnp.int32),
        pltpu.SemaphoreType.DMA,
        pltpu.SemaphoreType.DMA,
        pltpu.SemaphoreType.DMA,
        pltpu.SemaphoreType.DMA,
    ],
    compiler_params=pltpu.CompilerParams(needs_layout_passes=False),
)(_sc_gather_body)


_BN = 4096


def _mlp_body(pt_ref, nt_ref, w1t_ref, b1_ref, w2_ref, b2_ref,
              w3t_ref, b3_ref, o_ref):
    mm = functools.partial(jnp.dot, preferred_element_type=jnp.float32)
    w1t = w1t_ref[...]
    h = mm(w1t[:, :H], pt_ref[...]) + mm(w1t[:, H:], nt_ref[...])
    h = jnp.maximum(h + b1_ref[...].T, 0.0)
    h2 = lax.dot_general(w2_ref[...], h,
                         dimension_numbers=(((0,), (0,)), ((), ())),
                         preferred_element_type=jnp.float32)
    h2 = jnp.maximum(h2 + b2_ref[...].T, 0.0)
    o_ref[...] = mm(w3t_ref[...], h2) + b3_ref[...]


def _mlp(pt, nt, w1t, b1, w2, b2, w3t, b3):
    grid = (B // _BN,)
    return pl.pallas_call(
        _mlp_body,
        grid=grid,
        in_specs=[
            pl.BlockSpec((H, _BN), lambda i: (0, i)),
            pl.BlockSpec((H, _BN), lambda i: (0, i)),
            pl.BlockSpec((H, 2 * H), lambda i: (0, 0)),
            pl.BlockSpec((1, H), lambda i: (0, 0)),
            pl.BlockSpec((H, H), lambda i: (0, 0)),
            pl.BlockSpec((1, H), lambda i: (0, 0)),
            pl.BlockSpec((1, H), lambda i: (0, 0)),
            pl.BlockSpec((1, 1), lambda i: (0, 0)),
        ],
        out_specs=pl.BlockSpec((1, _BN), lambda i: (0, i)),
        out_shape=jax.ShapeDtypeStruct((1, B), jnp.float32),
    )(pt, nt, w1t, b1, w2, b2, w3t, b3)


def kernel(x, emb_proton, emb_neutron, W1, b1, W2, b2, W3, b3):
    x = x.astype(jnp.int32)
    idx_p = x[:, 0]
    idx_n = x[:, 1]
    tp = emb_proton.T
    tn = emb_neutron.T
    pt, nt = _sc_gather(tp, tn, tp[:, V - _TAIL:].reshape(-1),
                        tn[:, V - _TAIL:].reshape(-1), idx_p, idx_n)
    out_t = _mlp(pt, nt, W1.T, b1.reshape(1, H), W2, b2.reshape(1, H),
                 W3.reshape(1, H), b3.reshape(1, 1))
    return out_t.reshape(B, 1)
